# Initial kernel scaffold; baseline (speedup 1.0000x reference)
import jax
import jax.numpy as jnp
from jax.experimental import pallas as pl


def kernel(centernesses, cls_scores, points, features):
    raise NotImplementedError("write your pallas kernel here")


# speedup vs baseline: 29.7818x; 29.7818x over previous
import jax
import jax.numpy as jnp
from jax.experimental import pallas as pl
from jax.experimental.pallas import tpu as pltpu

_N = 20000
_NC = 18
_C = 128
_NCROSS = 2048
_K = 256
_L = 128
_R = 160
_NPAD = _R * _L
_BIG = 2 ** 30


def _sigmoid(x):
    return 1.0 / (1.0 + jnp.exp(-x))


def _main_kernel(cent_ref, cls_ref, px_ref, py_ref, pz_ref, feat_ref,
                 pts_out, feats_out, inds_out, cross_out,
                 dists_ref, mask_ref, bits_ref):
    idx2d = (jax.lax.broadcasted_iota(jnp.int32, (_R, _L), 0) * _L
             + jax.lax.broadcasted_iota(jnp.int32, (_R, _L), 1))
    lanes = jax.lax.broadcasted_iota(jnp.int32, (1, _L), 1)
    valid = idx2d < _N

    sig_c = _sigmoid(cent_ref[...])
    m = jnp.full((_R, _L), -1.0, dtype=jnp.float32)
    for c in range(_NC):
        m = jnp.maximum(m, _sigmoid(cls_ref[c]) * sig_c)
    scores = jnp.where(valid, m, 0.0)
    bits_ref[...] = jax.lax.bitcast_convert_type(scores, jnp.int32)

    def bs_body(_, lohi):
        lo, hi = lohi
        mid = jax.lax.div(lo + hi, jnp.int32(2))
        cnt = jnp.sum(jnp.where(bits_ref[...] >= mid, 1, 0))
        big = cnt >= _K
        return (jnp.where(big, mid, lo), jnp.where(big, hi, mid))

    lo0 = jnp.int32(1)
    hi0 = jnp.int32(0x3F800001)
    t, _ = jax.lax.fori_loop(0, 31, bs_body, (lo0, hi0))

    bits = bits_ref[...]
    gt = bits > t
    eq = bits == t
    cnt_gt = jnp.sum(jnp.where(gt, 1, 0))
    quota = _K - cnt_gt

    x = jnp.where(eq, 1, 0)
    eqi = x
    for sh in (1, 2, 4, 8, 16, 32, 64):
        x = x + jnp.pad(x, ((0, 0), (sh, 0)))[:, :_L]
    lane_inc = x
    row_tot = lane_inc[:, _L - 1:_L]
    y = row_tot
    for sh in (1, 2, 4, 8, 16, 32, 64, 128):
        y = y + jnp.pad(y, ((sh, 0), (0, 0)))[:_R, :]
    rank = (y - row_tot) + (lane_inc - eqi)
    selected = gt | (eq & (rank < quota))
    mask_ref[...] = jnp.where(selected, 1, 0)

    def ext_body(j, carry):
        msk = mask_ref[...]
        idx = jnp.min(jnp.where(msk != 0, idx2d, _BIG))
        mask_ref[...] = jnp.where(idx2d == idx, 0, msk)
        inds_out[j] = idx
        feats_out[pl.ds(j, 1), :] = feat_ref[pl.ds(idx, 1), :]
        r = jax.lax.div(idx, jnp.int32(_L))
        cc = idx - r * _L
        sel = lanes == cc
        vx = jnp.sum(jnp.where(sel, px_ref[pl.ds(r, 1), :], 0.0))
        vy = jnp.sum(jnp.where(sel, py_ref[pl.ds(r, 1), :], 0.0))
        vz = jnp.sum(jnp.where(sel, pz_ref[pl.ds(r, 1), :], 0.0))
        pts_out[pl.ds(j, 1), :] = jnp.concatenate(
            [vx.reshape(1, 1), vy.reshape(1, 1), vz.reshape(1, 1)], axis=1)
        return carry

    jax.lax.fori_loop(0, _K, ext_body, 0)

    dists_ref[...] = jnp.where(valid, jnp.inf, -jnp.inf)
    cross_out[pl.ds(0, 1), :] = feat_ref[pl.ds(0, 1), :]
    sel0 = lanes == 0
    lx0 = jnp.sum(jnp.where(sel0, px_ref[0:1, :], 0.0))
    ly0 = jnp.sum(jnp.where(sel0, py_ref[0:1, :], 0.0))
    lz0 = jnp.sum(jnp.where(sel0, pz_ref[0:1, :], 0.0))

    def fps_body(i, carry):
        lx, ly, lz = carry
        dx = px_ref[...] - lx
        dy = py_ref[...] - ly
        dz = pz_ref[...] - lz
        d = (dx * dx + dy * dy) + dz * dz
        nd = jnp.minimum(dists_ref[...], d)
        dists_ref[...] = nd
        mx = jnp.max(nd)
        nxt = jnp.min(jnp.where(nd == mx, idx2d, _BIG))
        cross_out[pl.ds(i, 1), :] = feat_ref[pl.ds(nxt, 1), :]
        r = jax.lax.div(nxt, jnp.int32(_L))
        cc = nxt - r * _L
        sel = lanes == cc
        nlx = jnp.sum(jnp.where(sel, px_ref[pl.ds(r, 1), :], 0.0))
        nly = jnp.sum(jnp.where(sel, py_ref[pl.ds(r, 1), :], 0.0))
        nlz = jnp.sum(jnp.where(sel, pz_ref[pl.ds(r, 1), :], 0.0))
        return (nlx, nly, nlz)

    jax.lax.fori_loop(1, _NCROSS, fps_body, (lx0, ly0, lz0))


def kernel(centernesses, cls_scores, points, features):
    pad = _NPAD - _N
    cent2d = jnp.pad(centernesses, (0, pad)).reshape(_R, _L)
    cls_t = jnp.pad(cls_scores.T, ((0, 0), (0, pad))).reshape(_NC, _R, _L)
    pts_t = jnp.pad(points.T, ((0, 0), (0, pad))).reshape(3, _R, _L)
    px, py, pz = pts_t[0], pts_t[1], pts_t[2]

    pts, feats, inds, cross = pl.pallas_call(
        _main_kernel,
        out_shape=[
            jax.ShapeDtypeStruct((_K, 3), jnp.float32),
            jax.ShapeDtypeStruct((_K, _C), jnp.float32),
            jax.ShapeDtypeStruct((_K,), jnp.int32),
            jax.ShapeDtypeStruct((_NCROSS, _C), jnp.float32),
        ],
        out_specs=[
            pl.BlockSpec(memory_space=pltpu.VMEM),
            pl.BlockSpec(memory_space=pltpu.VMEM),
            pl.BlockSpec(memory_space=pltpu.SMEM),
            pl.BlockSpec(memory_space=pltpu.VMEM),
        ],
        scratch_shapes=[
            pltpu.VMEM((_R, _L), jnp.float32),
            pltpu.VMEM((_R, _L), jnp.int32),
            pltpu.VMEM((_R, _L), jnp.int32),
        ],
    )(cent2d, cls_t, px, py, pz, features)

    return (pts[None], feats[None], inds[None], cross[None])


# speedup vs baseline: 54.2012x; 1.8199x over previous
import jax
import jax.numpy as jnp
from jax.experimental import pallas as pl
from jax.experimental.pallas import tpu as pltpu

_N = 20000
_NC = 18
_C = 128
_NCROSS = 2048
_K = 256
_L = 128
_R = 160
_NPAD = _R * _L
_BIG = 2 ** 30


def _sigmoid(x):
    return 1.0 / (1.0 + jnp.exp(-x))


def _main_kernel(stk_ref, feat_ref, psm_ref,
                 pts_out, feats_out, inds_out, cross_out,
                 mask_ref, bits_ref):
    idx2d = (jax.lax.broadcasted_iota(jnp.int32, (_R, _L), 0) * _L
             + jax.lax.broadcasted_iota(jnp.int32, (_R, _L), 1))
    valid = idx2d < _N

    sig_c = _sigmoid(stk_ref[0])
    m = jnp.full((_R, _L), -1.0, dtype=jnp.float32)
    for c in range(_NC):
        m = jnp.maximum(m, _sigmoid(stk_ref[1 + c]) * sig_c)
    scores = jnp.where(valid, m, 0.0)
    bits_ref[...] = jax.lax.bitcast_convert_type(scores, jnp.int32)

    def bs_body(_, lohi):
        lo, hi = lohi
        mid = jax.lax.div(lo + hi, jnp.int32(2))
        cnt = jnp.sum(jnp.where(bits_ref[...] >= mid, 1, 0))
        big = cnt >= _K
        return (jnp.where(big, mid, lo), jnp.where(big, hi, mid))

    lo0 = jnp.int32(1)
    hi0 = jnp.int32(0x3F800001)
    t, _ = jax.lax.fori_loop(0, 31, bs_body, (lo0, hi0))

    bits = bits_ref[...]
    gt = bits > t
    eq = bits == t
    cnt_gt = jnp.sum(jnp.where(gt, 1, 0))
    quota = _K - cnt_gt

    x = jnp.where(eq, 1, 0)
    eqi = x
    for sh in (1, 2, 4, 8, 16, 32, 64):
        x = x + jnp.pad(x, ((0, 0), (sh, 0)))[:, :_L]
    lane_inc = x
    row_tot = lane_inc[:, _L - 1:_L]
    y = row_tot
    for sh in (1, 2, 4, 8, 16, 32, 64, 128):
        y = y + jnp.pad(y, ((sh, 0), (0, 0)))[:_R, :]
    rank = (y - row_tot) + (lane_inc - eqi)
    selected = gt | (eq & (rank < quota))
    mask_ref[...] = jnp.where(selected, 1, 0).reshape(_R // 8, 8, _L)

    idx3 = idx2d.reshape(_R // 8, 8, _L)

    def _fold(planes, take_fn, stop):
        w = planes[0].shape[0]
        while w > stop:
            h = w // 2
            a = tuple(p[0:h] for p in planes)
            b = tuple(p[h:2 * h] for p in planes)
            take = take_fn(b, a)
            comb = tuple(jnp.where(take, pb, pa) for pb, pa in zip(b, a))
            if w % 2:
                planes = tuple(
                    jnp.concatenate([pc, p[2 * h:w]], axis=0)
                    for pc, p in zip(comb, planes))
                w = h + 1
            else:
                planes = comb
                w = h
        return planes

    def _slane_bfly(planes, take_fn):
        for sh in (4, 2, 1):
            r = tuple(pltpu.roll(p, sh, 0) for p in planes)
            take = take_fn(r, planes)
            planes = tuple(jnp.where(take, pr, pp)
                           for pr, pp in zip(r, planes))
        return planes

    def _argreduce(planes, take_fn):
        planes = _fold(planes, take_fn, 1)
        planes = tuple(p.reshape(8, _L) for p in planes)
        planes = _slane_bfly(planes, take_fn)
        planes = tuple(jnp.swapaxes(p, 0, 1) for p in planes)
        planes = _fold(planes, take_fn, 8)
        planes = _slane_bfly(planes, take_fn)
        return tuple(p[0, 0] for p in planes)

    px3 = stk_ref[1 + _NC].reshape(_R // 8, 8, _L)
    py3 = stk_ref[2 + _NC].reshape(_R // 8, 8, _L)
    pz3 = stk_ref[3 + _NC].reshape(_R // 8, 8, _L)

    def _take_min_idx(b, a):
        return b[0] < a[0]

    def ext_body(j, carry):
        msk = mask_ref[...]
        cand = jnp.where(msk != 0, idx3, _BIG)
        (idx,) = _argreduce((cand,), _take_min_idx)
        mask_ref[...] = jnp.where(idx3 == idx, 0, msk)
        inds_out[j] = idx
        feats_out[pl.ds(j, 1), :] = feat_ref[pl.ds(idx, 1), :]
        base = idx * 3
        pts_out[pl.ds(j, 1), :] = jnp.concatenate(
            [psm_ref[base].reshape(1, 1), psm_ref[base + 1].reshape(1, 1),
             psm_ref[base + 2].reshape(1, 1)], axis=1)
        return carry

    jax.lax.fori_loop(0, _K, ext_body, 0)

    cross_out[pl.ds(0, 1), :] = feat_ref[pl.ds(0, 1), :]
    lx0 = psm_ref[0]
    ly0 = psm_ref[1]
    lz0 = psm_ref[2]
    d0 = jnp.where(idx3 < _N, jnp.inf, -jnp.inf)

    def _take_max_val_first(b, a):
        return (b[0] > a[0]) | ((b[0] == a[0]) & (b[1] < a[1]))

    def fps_iter(i, lx, ly, lz, dists):
        dx = px3 - lx
        dy = py3 - ly
        dz = pz3 - lz
        d = (dx * dx + dy * dy) + dz * dz
        nd = jnp.minimum(dists, d)
        _, nxt = _argreduce((nd, idx3), _take_max_val_first)
        cross_out[pl.ds(i, 1), :] = feat_ref[pl.ds(nxt, 1), :]
        base = nxt * 3
        return (psm_ref[base], psm_ref[base + 1], psm_ref[base + 2], nd)

    def fps_pair(k, carry):
        i1 = 2 * k + 1
        s1 = fps_iter(i1, *carry)
        return fps_iter(i1 + 1, *s1)

    carry = jax.lax.fori_loop(0, (_NCROSS - 2) // 2, fps_pair,
                              (lx0, ly0, lz0, d0))
    fps_iter(_NCROSS - 1, *carry)


def kernel(centernesses, cls_scores, points, features):
    pad = _NPAD - _N
    stack = jnp.concatenate(
        [centernesses[None, :], cls_scores.T, points.T], axis=0)
    stack = jnp.pad(stack, ((0, 0), (0, pad))).reshape(4 + _NC, _R, _L)
    psm = points.reshape(_N * 3)

    pts, feats, inds, cross = pl.pallas_call(
        _main_kernel,
        in_specs=[
            pl.BlockSpec(memory_space=pltpu.VMEM),
            pl.BlockSpec(memory_space=pltpu.VMEM),
            pl.BlockSpec(memory_space=pltpu.SMEM),
        ],
        out_shape=[
            jax.ShapeDtypeStruct((_K, 3), jnp.float32),
            jax.ShapeDtypeStruct((_K, _C), jnp.float32),
            jax.ShapeDtypeStruct((_K,), jnp.int32),
            jax.ShapeDtypeStruct((_NCROSS, _C), jnp.float32),
        ],
        out_specs=[
            pl.BlockSpec(memory_space=pltpu.VMEM),
            pl.BlockSpec(memory_space=pltpu.VMEM),
            pl.BlockSpec(memory_space=pltpu.SMEM),
            pl.BlockSpec(memory_space=pltpu.VMEM),
        ],
        scratch_shapes=[
            pltpu.VMEM((_R // 8, 8, _L), jnp.int32),
            pltpu.VMEM((_R, _L), jnp.int32),
        ],
    )(stack, features, psm)

    return (pts[None], feats[None], inds[None], cross[None])


# speedup vs baseline: 60.4721x; 1.1157x over previous
import jax
import jax.numpy as jnp
from jax.experimental import pallas as pl
from jax.experimental.pallas import tpu as pltpu

_N = 20000
_NC = 18
_C = 128
_NCROSS = 2048
_K = 256
_L = 128
_R = 160
_NPAD = _R * _L
_BIG = 2 ** 30


def _sigmoid(x):
    return 1.0 / (1.0 + jnp.exp(-x))


def _main_kernel(stk_ref, feat_ref, psm_ref,
                 pts_out, feats_out, inds_out, cross_out,
                 mask_ref, bits_ref):
    idx2d = (jax.lax.broadcasted_iota(jnp.int32, (_R, _L), 0) * _L
             + jax.lax.broadcasted_iota(jnp.int32, (_R, _L), 1))
    valid = idx2d < _N

    sig_c = _sigmoid(stk_ref[0])
    m = jnp.full((_R, _L), -1.0, dtype=jnp.float32)
    for c in range(_NC):
        m = jnp.maximum(m, _sigmoid(stk_ref[1 + c]) * sig_c)
    scores = jnp.where(valid, m, 0.0)
    bits_ref[...] = jax.lax.bitcast_convert_type(scores, jnp.int32)

    def bs_body(_, lohi):
        lo, hi = lohi
        mid = jax.lax.div(lo + hi, jnp.int32(2))
        cnt = jnp.sum(jnp.where(bits_ref[...] >= mid, 1, 0))
        big = cnt >= _K
        return (jnp.where(big, mid, lo), jnp.where(big, hi, mid))

    lo0 = jnp.int32(1)
    hi0 = jnp.int32(0x3F800001)
    t, _ = jax.lax.fori_loop(0, 31, bs_body, (lo0, hi0))

    bits = bits_ref[...]
    gt = bits > t
    eq = bits == t
    cnt_gt = jnp.sum(jnp.where(gt, 1, 0))
    quota = _K - cnt_gt

    x = jnp.where(eq, 1, 0)
    eqi = x
    for sh in (1, 2, 4, 8, 16, 32, 64):
        x = x + jnp.pad(x, ((0, 0), (sh, 0)))[:, :_L]
    lane_inc = x
    row_tot = lane_inc[:, _L - 1:_L]
    y = row_tot
    for sh in (1, 2, 4, 8, 16, 32, 64, 128):
        y = y + jnp.pad(y, ((sh, 0), (0, 0)))[:_R, :]
    rank = (y - row_tot) + (lane_inc - eqi)
    selected = gt | (eq & (rank < quota))
    mask_ref[...] = jnp.where(selected, 1, 0).reshape(_R // 8, 8, _L)

    idx3 = idx2d.reshape(_R // 8, 8, _L)

    def _fold_ew(x, op, stop):
        w = x.shape[0]
        while w > stop:
            h = w // 2
            comb = op(x[0:h], x[h:2 * h])
            if w % 2:
                x = jnp.concatenate([comb, x[2 * h:w]], axis=0)
                w = h + 1
            else:
                x = comb
                w = h
        return x

    def _slane_bfly_ew(x, op):
        for sh in (4, 2, 1):
            x = op(x, pltpu.roll(x, sh, 0))
        return x

    def _argmax_first(val, idxp):
        v8 = _slane_bfly_ew(
            _fold_ew(val, jnp.maximum, 1).reshape(8, _L), jnp.maximum)
        cand = jnp.where(val == v8[None], idxp, _BIG)
        i8 = _slane_bfly_ew(
            _fold_ew(cand, jnp.minimum, 1).reshape(8, _L), jnp.minimum)
        vt = jnp.swapaxes(v8, 0, 1)
        it = jnp.swapaxes(i8, 0, 1)
        mt = _slane_bfly_ew(_fold_ew(vt, jnp.maximum, 8), jnp.maximum)
        candt = jnp.where(vt == mt[0:1], it, _BIG)
        imin = _slane_bfly_ew(_fold_ew(candt, jnp.minimum, 8), jnp.minimum)
        return imin[0, 0]

    def _argmin_idx(cand):
        i8 = _slane_bfly_ew(
            _fold_ew(cand, jnp.minimum, 1).reshape(8, _L), jnp.minimum)
        it = jnp.swapaxes(i8, 0, 1)
        imin = _slane_bfly_ew(_fold_ew(it, jnp.minimum, 8), jnp.minimum)
        return imin[0, 0]

    px3 = stk_ref[1 + _NC].reshape(_R // 8, 8, _L)
    py3 = stk_ref[2 + _NC].reshape(_R // 8, 8, _L)
    pz3 = stk_ref[3 + _NC].reshape(_R // 8, 8, _L)

    def ext_body(j, carry):
        msk = mask_ref[...]
        cand = jnp.where(msk != 0, idx3, _BIG)
        idx = _argmin_idx(cand)
        mask_ref[...] = jnp.where(idx3 == idx, 0, msk)
        inds_out[j] = idx
        feats_out[pl.ds(j, 1), :] = feat_ref[pl.ds(idx, 1), :]
        base = idx * 3
        pts_out[pl.ds(j, 1), :] = jnp.concatenate(
            [psm_ref[base].reshape(1, 1), psm_ref[base + 1].reshape(1, 1),
             psm_ref[base + 2].reshape(1, 1)], axis=1)
        return carry

    jax.lax.fori_loop(0, _K, ext_body, 0)

    cross_out[pl.ds(0, 1), :] = feat_ref[pl.ds(0, 1), :]
    lx0 = psm_ref[0]
    ly0 = psm_ref[1]
    lz0 = psm_ref[2]
    d0 = jnp.where(idx3 < _N, jnp.inf, -jnp.inf)

    def fps_iter(i, lx, ly, lz, dists):
        dx = px3 - lx
        dy = py3 - ly
        dz = pz3 - lz
        d = (dx * dx + dy * dy) + dz * dz
        nd = jnp.minimum(dists, d)
        nxt = _argmax_first(nd, idx3)
        cross_out[pl.ds(i, 1), :] = feat_ref[pl.ds(nxt, 1), :]
        base = nxt * 3
        return (psm_ref[base], psm_ref[base + 1], psm_ref[base + 2], nd)

    def fps_pair(k, carry):
        i1 = 2 * k + 1
        s1 = fps_iter(i1, *carry)
        return fps_iter(i1 + 1, *s1)

    carry = jax.lax.fori_loop(0, (_NCROSS - 2) // 2, fps_pair,
                              (lx0, ly0, lz0, d0))
    fps_iter(_NCROSS - 1, *carry)


def kernel(centernesses, cls_scores, points, features):
    pad = _NPAD - _N
    stack = jnp.concatenate(
        [centernesses[None, :], cls_scores.T, points.T], axis=0)
    stack = jnp.pad(stack, ((0, 0), (0, pad))).reshape(4 + _NC, _R, _L)
    psm = points.reshape(_N * 3)

    pts, feats, inds, cross = pl.pallas_call(
        _main_kernel,
        in_specs=[
            pl.BlockSpec(memory_space=pltpu.VMEM),
            pl.BlockSpec(memory_space=pltpu.VMEM),
            pl.BlockSpec(memory_space=pltpu.SMEM),
        ],
        out_shape=[
            jax.ShapeDtypeStruct((_K, 3), jnp.float32),
            jax.ShapeDtypeStruct((_K, _C), jnp.float32),
            jax.ShapeDtypeStruct((_K,), jnp.int32),
            jax.ShapeDtypeStruct((_NCROSS, _C), jnp.float32),
        ],
        out_specs=[
            pl.BlockSpec(memory_space=pltpu.VMEM),
            pl.BlockSpec(memory_space=pltpu.VMEM),
            pl.BlockSpec(memory_space=pltpu.SMEM),
            pl.BlockSpec(memory_space=pltpu.VMEM),
        ],
        scratch_shapes=[
            pltpu.VMEM((_R // 8, 8, _L), jnp.int32),
            pltpu.VMEM((_R, _L), jnp.int32),
        ],
    )(stack, features, psm)

    return (pts[None], feats[None], inds[None], cross[None])
